# speedup vs baseline: 1.0441x; 1.0441x over previous
import functools

import jax
import jax.numpy as jnp
from jax.experimental import pallas as pl
from jax.experimental.pallas import tpu as pltpu

_N = 100000
_D = 128
_R = 14
_E = 40000
_L = 4
_NP = 100352


def _h_mm_body(x_ref, w_ref, o_ref):
    o_ref[...] = jnp.dot(x_ref[...], w_ref[...],
                         preferred_element_type=jnp.float32)


def _h_matmul(x, wcat):
    bn = 512
    return pl.pallas_call(
        _h_mm_body,
        grid=(_NP // bn,),
        in_specs=[pl.BlockSpec((bn, _D), lambda i: (i, 0)),
                  pl.BlockSpec((_D, 15 * _D), lambda i: (0, 0))],
        out_specs=pl.BlockSpec((bn, 15 * _D), lambda i: (i, 0)),
        out_shape=jax.ShapeDtypeStruct((_NP, 15 * _D), jnp.float32),
    )(x, wcat)


def _post_body(t_ref, r_ref, w2_ref, g1w_ref, g1b_ref, g2w_ref, g2b_ref,
               o_ref):
    t = t_ref[...]
    mu = jnp.mean(t, axis=-1, keepdims=True)
    var = jnp.mean((t - mu) ** 2, axis=-1, keepdims=True)
    x = (t - mu) * jax.lax.rsqrt(var + 1e-5) * g1w_ref[...] + g1b_ref[...]
    x = jnp.maximum(x, 0.0)
    y = jnp.dot(x, w2_ref[...], preferred_element_type=jnp.float32)
    mu2 = jnp.mean(y, axis=-1, keepdims=True)
    var2 = jnp.mean((y - mu2) ** 2, axis=-1, keepdims=True)
    y = (y - mu2) * jax.lax.rsqrt(var2 + 1e-5) * g2w_ref[...] + g2b_ref[...]
    o_ref[...] = jnp.maximum(y + r_ref[...], 0.0)


def _post_stage(temp, res, w2t, g1w, g1b, g2w, g2b):
    bn = 1024
    vec = pl.BlockSpec((1, _D), lambda i: (0, 0))
    return pl.pallas_call(
        _post_body,
        grid=(_NP // bn,),
        in_specs=[pl.BlockSpec((bn, _D), lambda i: (i, 0)),
                  pl.BlockSpec((bn, _D), lambda i: (i, 0)),
                  pl.BlockSpec((_D, _D), lambda i: (0, 0)),
                  vec, vec, vec, vec],
        out_specs=pl.BlockSpec((bn, _D), lambda i: (i, 0)),
        out_shape=jax.ShapeDtypeStruct((_NP, _D), jnp.float32),
    )(temp, res, w2t, g1w, g1b, g2w, g2b)


def kernel(feat, W_ctr, W_rel, gn1_w, gn1_b, W_ctr2, gn2_w, gn2_b,
           u_idx, v_idx):
    feat_p = jnp.pad(feat, ((0, _NP - _N), (0, 0)))
    wk_all = jnp.concatenate([W_ctr[:, None], W_rel], axis=1)
    wcat = wk_all.transpose(0, 3, 1, 2).reshape(_L, _D, 15 * _D)
    w2t = W_ctr2.transpose(0, 2, 1)

    src_all = (v_idx * 15 + (jnp.arange(_R, dtype=jnp.int32) + 1)[:, None]
               ).reshape(-1)
    u_flat = u_idx.reshape(-1)

    res = feat_p
    x = feat_p
    for i in range(_L):
        h = _h_matmul(x, wcat[i])
        temp = h[:, :_D]
        msg = h.reshape(_NP * 15, _D)[src_all]
        temp = temp.at[u_flat].add(msg)
        x = _post_stage(temp, res,
                        w2t[i],
                        gn1_w[i][None, :], gn1_b[i][None, :],
                        gn2_w[i][None, :], gn2_b[i][None, :])
        res = x
    return x[:_N]
